# baseline (speedup 1.0000x reference)
import functools

import jax
import jax.numpy as jnp
from jax import lax
from jax.experimental import pallas as pl
from jax.experimental.pallas import tpu as pltpu
from jax.experimental.pallas import tpu_sc as plsc

N = 100000
D = 128
S = 512
NC = 2
NS = 16
NW = NC * NS
RPW = N // NW
C = 125
NCH = RPW // C
K = 12
RPT = S // NS
VPR = D // 16


NBUF = 4


def _sc_body(x_hbm, b_hbm, out_hbm, idx_v, xb0, xb1, xb2, xb3, zbuf,
             acc_sh, fs0, fs1, fs2, fs3, ss0, ss1, ss2, ss3):
    c = lax.axis_index("c")
    s = lax.axis_index("s")
    wid = s * NC + c
    base = wid * RPW
    xb = [xb0, xb1, xb2, xb3]
    fs = [fs0, fs1, fs2, fs3]
    ss = [ss0, ss1, ss2, ss3]

    def _fetch(j, b):
        pltpu.async_copy(x_hbm.at[pl.ds(base + j * C, C)], xb[b], fs[b])

    def _fetch_wait(b):
        pltpu.make_async_copy(x_hbm.at[pl.ds(base, C)], xb[b], fs[b]).wait()

    def _scat(j, b):
        pltpu.async_copy(xb[b], acc_sh.at[idx_v.at[j]], ss[b], add=True)

    def _scat_wait(b):
        pltpu.make_async_copy(xb[b], acc_sh.at[idx_v.at[0]], ss[b]).wait()

    for b in range(NBUF):
        _fetch(b, b)

    zrow = jnp.zeros((16,), jnp.float32)

    def _zero_row(i, carry):
        for cc in range(VPR):
            zbuf[i, pl.ds(cc * 16, 16)] = zrow
        return carry

    lax.fori_loop(0, RPT, _zero_row, 0)
    for k in range(K):
        pltpu.sync_copy(zbuf, acc_sh.at[pl.ds(k * S + s * RPT, RPT)])

    pltpu.sync_copy(b_hbm.at[pl.ds(wid * NCH, NCH)], idx_v)
    plsc.subcore_barrier()

    def _grp(g, carry):
        for b in range(NBUF):
            j = g * NBUF + b
            bf = (b + NBUF - 1) % NBUF

            @pl.when(jnp.logical_and(j >= 1, j + NBUF - 1 < NCH))
            def _():
                _scat_wait(bf)
                _fetch(j + NBUF - 1, bf)

            _fetch_wait(b)
            _scat(j, b)
        return carry

    lax.fori_loop(0, NCH // NBUF, _grp, 0)
    _fetch_wait(0)
    _scat(NCH - 1, 0)
    for b in [1, 2, 3, 0]:
        _scat_wait(b)
    plsc.subcore_barrier()

    pltpu.sync_copy(acc_sh.at[pl.ds(s * RPT, RPT)], zbuf)
    _rep = [xb0, xb1]
    for k in range(1, K):
        rb = _rep[k % 2]
        pltpu.async_copy(acc_sh.at[pl.ds(k * S + s * RPT, RPT)],
                         rb.at[pl.ds(0, RPT)], fs[k % 2])
        if k > 1:
            pb = _rep[(k - 1) % 2]
            pltpu.make_async_copy(acc_sh.at[pl.ds(0, RPT)],
                                  pb.at[pl.ds(0, RPT)], fs[(k - 1) % 2]).wait()

            def _add_row(i, carry, _pb=pb):
                for cc in range(VPR):
                    zbuf[i, pl.ds(cc * 16, 16)] = (
                        zbuf[i, pl.ds(cc * 16, 16)]
                        + _pb[i, pl.ds(cc * 16, 16)])
                return carry

            lax.fori_loop(0, RPT, _add_row, 0)
    pltpu.make_async_copy(acc_sh.at[pl.ds(0, RPT)],
                          _rep[(K - 1) % 2].at[pl.ds(0, RPT)],
                          fs[(K - 1) % 2]).wait()

    def _add_last(i, carry):
        for cc in range(VPR):
            zbuf[i, pl.ds(cc * 16, 16)] = (
                zbuf[i, pl.ds(cc * 16, 16)]
                + _rep[(K - 1) % 2][i, pl.ds(cc * 16, 16)])
        return carry

    lax.fori_loop(0, RPT, _add_last, 0)
    pltpu.sync_copy(zbuf, out_hbm.at[c, pl.ds(s * RPT, RPT)])


_sc_call = functools.partial(
    pl.kernel,
    out_type=jax.ShapeDtypeStruct((NC, S, D), jnp.float32),
    mesh=plsc.VectorSubcoreMesh(core_axis_name="c", subcore_axis_name="s"),
    scratch_types=[
        pltpu.VMEM((NCH, C), jnp.int32),
        pltpu.VMEM((C, D), jnp.float32),
        pltpu.VMEM((C, D), jnp.float32),
        pltpu.VMEM((C, D), jnp.float32),
        pltpu.VMEM((C, D), jnp.float32),
        pltpu.VMEM((RPT, D), jnp.float32),
        pltpu.VMEM_SHARED((K * S, D), jnp.float32),
        pltpu.SemaphoreType.DMA,
        pltpu.SemaphoreType.DMA,
        pltpu.SemaphoreType.DMA,
        pltpu.SemaphoreType.DMA,
        pltpu.SemaphoreType.DMA,
        pltpu.SemaphoreType.DMA,
        pltpu.SemaphoreType.DMA,
        pltpu.SemaphoreType.DMA,
    ],
    compiler_params=pltpu.CompilerParams(use_tc_tiling_on_sc=False),
)(_sc_body)


def _combine_body(p_ref, o_ref):
    o_ref[...] = p_ref[0] + p_ref[1]


def kernel(x, batch):
    spread = (jnp.arange(C, dtype=jnp.int32) % K) * S
    b2 = batch.astype(jnp.int32).reshape(N // C, C) + spread[None, :]
    partials = _sc_call(x, b2)
    return pl.pallas_call(
        _combine_body,
        out_shape=jax.ShapeDtypeStruct((S, D), jnp.float32),
    )(partials)


# speedup vs baseline: 1.0283x; 1.0283x over previous
import functools

import jax
import jax.numpy as jnp
from jax import lax
from jax.experimental import pallas as pl
from jax.experimental.pallas import tpu as pltpu
from jax.experimental.pallas import tpu_sc as plsc

N = 100000
D = 128
S = 512
NC = 2
NS = 16
NW = NC * NS
RPW = N // NW
C = 125
NCH = RPW // C
K = 8
RPT = S // NS
VPR = D // 16


NBUF = 4


def _sc_body(x_hbm, b_hbm, out_hbm, idx_v, xb0, xb1, xb2, xb3, zbuf,
             acc_sh, fs0, fs1, fs2, fs3, ss0, ss1, ss2, ss3, zs, isem):
    c = lax.axis_index("c")
    s = lax.axis_index("s")
    wid = s * NC + c
    base = wid * RPW
    xb = [xb0, xb1, xb2, xb3]
    fs = [fs0, fs1, fs2, fs3]
    ss = [ss0, ss1, ss2, ss3]

    def _fetch(j, b):
        pltpu.async_copy(x_hbm.at[pl.ds(base + j * C, C)], xb[b], fs[b])

    def _fetch_wait(b):
        pltpu.make_async_copy(x_hbm.at[pl.ds(base, C)], xb[b], fs[b]).wait()

    def _scat(j, b):
        pltpu.async_copy(xb[b], acc_sh.at[idx_v.at[j]], ss[b], add=True)

    def _scat_wait(b):
        pltpu.make_async_copy(xb[b], acc_sh.at[idx_v.at[0]], ss[b]).wait()

    for b in range(NBUF):
        _fetch(b, b)
    pltpu.async_copy(b_hbm.at[pl.ds(wid * NCH, NCH)], idx_v, isem)

    zrow = jnp.zeros((16,), jnp.float32)

    def _zero_row(i, carry):
        for cc in range(VPR):
            zbuf[i, pl.ds(cc * 16, 16)] = zrow
        return carry

    lax.fori_loop(0, RPT, _zero_row, 0)
    for k in range(K):
        pltpu.async_copy(zbuf, acc_sh.at[pl.ds(k * S + s * RPT, RPT)], zs)
    for k in range(K):
        pltpu.make_async_copy(zbuf, acc_sh.at[pl.ds(s * RPT, RPT)], zs).wait()
    pltpu.make_async_copy(b_hbm.at[pl.ds(0, NCH)], idx_v, isem).wait()
    plsc.subcore_barrier()

    def _grp(g, carry):
        for b in range(NBUF):
            j = g * NBUF + b
            bf = (b + NBUF - 1) % NBUF

            @pl.when(jnp.logical_and(j >= 1, j + NBUF - 1 < NCH))
            def _():
                _scat_wait(bf)
                _fetch(j + NBUF - 1, bf)

            _fetch_wait(b)
            _scat(j, b)
        return carry

    lax.fori_loop(0, NCH // NBUF, _grp, 0)
    _fetch_wait(0)
    _scat(NCH - 1, 0)
    for b in [1, 2, 3, 0]:
        _scat_wait(b)
    plsc.subcore_barrier()

    pltpu.sync_copy(acc_sh.at[pl.ds(s * RPT, RPT)], zbuf)
    _rep = [xb0, xb1]
    for k in range(1, K):
        rb = _rep[k % 2]
        pltpu.async_copy(acc_sh.at[pl.ds(k * S + s * RPT, RPT)],
                         rb.at[pl.ds(0, RPT)], fs[k % 2])
        if k > 1:
            pb = _rep[(k - 1) % 2]
            pltpu.make_async_copy(acc_sh.at[pl.ds(0, RPT)],
                                  pb.at[pl.ds(0, RPT)], fs[(k - 1) % 2]).wait()

            def _add_row(i, carry, _pb=pb):
                for cc in range(VPR):
                    zbuf[i, pl.ds(cc * 16, 16)] = (
                        zbuf[i, pl.ds(cc * 16, 16)]
                        + _pb[i, pl.ds(cc * 16, 16)])
                return carry

            lax.fori_loop(0, RPT, _add_row, 0)
    pltpu.make_async_copy(acc_sh.at[pl.ds(0, RPT)],
                          _rep[(K - 1) % 2].at[pl.ds(0, RPT)],
                          fs[(K - 1) % 2]).wait()

    def _add_last(i, carry):
        for cc in range(VPR):
            zbuf[i, pl.ds(cc * 16, 16)] = (
                zbuf[i, pl.ds(cc * 16, 16)]
                + _rep[(K - 1) % 2][i, pl.ds(cc * 16, 16)])
        return carry

    lax.fori_loop(0, RPT, _add_last, 0)
    pltpu.sync_copy(zbuf, out_hbm.at[c, pl.ds(s * RPT, RPT)])


_sc_call = functools.partial(
    pl.kernel,
    out_type=jax.ShapeDtypeStruct((NC, S, D), jnp.float32),
    mesh=plsc.VectorSubcoreMesh(core_axis_name="c", subcore_axis_name="s"),
    scratch_types=[
        pltpu.VMEM((NCH, C), jnp.int32),
        pltpu.VMEM((C, D), jnp.float32),
        pltpu.VMEM((C, D), jnp.float32),
        pltpu.VMEM((C, D), jnp.float32),
        pltpu.VMEM((C, D), jnp.float32),
        pltpu.VMEM((RPT, D), jnp.float32),
        pltpu.VMEM_SHARED((K * S, D), jnp.float32),
        pltpu.SemaphoreType.DMA,
        pltpu.SemaphoreType.DMA,
        pltpu.SemaphoreType.DMA,
        pltpu.SemaphoreType.DMA,
        pltpu.SemaphoreType.DMA,
        pltpu.SemaphoreType.DMA,
        pltpu.SemaphoreType.DMA,
        pltpu.SemaphoreType.DMA,
        pltpu.SemaphoreType.DMA,
        pltpu.SemaphoreType.DMA,
    ],
    compiler_params=pltpu.CompilerParams(use_tc_tiling_on_sc=False),
)(_sc_body)


def _combine_body(p_ref, o_ref):
    o_ref[...] = p_ref[0] + p_ref[1]


def kernel(x, batch):
    spread = (jnp.arange(C, dtype=jnp.int32) % K) * S
    b2 = batch.astype(jnp.int32).reshape(N // C, C) + spread[None, :]
    partials = _sc_call(x, b2)
    return pl.pallas_call(
        _combine_body,
        out_shape=jax.ShapeDtypeStruct((S, D), jnp.float32),
    )(partials)


# speedup vs baseline: 1.0565x; 1.0274x over previous
import functools

import jax
import jax.numpy as jnp
from jax import lax
from jax.experimental import pallas as pl
from jax.experimental.pallas import tpu as pltpu
from jax.experimental.pallas import tpu_sc as plsc

N = 100000
D = 128
S = 512
NC = 2
NS = 16
NW = NC * NS
RPW = N // NW
C = 125
NCH = RPW // C
K = 4
RPT = S // NS
VPR = D // 16


NBUF = 4


def _sc_body(x_hbm, b_hbm, out_hbm, idx_v, xb0, xb1, xb2, xb3, zbuf,
             acc_sh, fs0, fs1, fs2, fs3, ss0, ss1, ss2, ss3, zs, isem):
    c = lax.axis_index("c")
    s = lax.axis_index("s")
    wid = s * NC + c
    base = wid * RPW
    xb = [xb0, xb1, xb2, xb3]
    fs = [fs0, fs1, fs2, fs3]
    ss = [ss0, ss1, ss2, ss3]

    def _fetch(j, b):
        pltpu.async_copy(x_hbm.at[pl.ds(base + j * C, C)], xb[b], fs[b])

    def _fetch_wait(b):
        pltpu.make_async_copy(x_hbm.at[pl.ds(base, C)], xb[b], fs[b]).wait()

    def _scat(j, b):
        pltpu.async_copy(xb[b], acc_sh.at[idx_v.at[j]], ss[b], add=True)

    def _scat_wait(b):
        pltpu.make_async_copy(xb[b], acc_sh.at[idx_v.at[0]], ss[b]).wait()

    for b in range(NBUF):
        _fetch(b, b)
    pltpu.async_copy(b_hbm.at[pl.ds(wid * NCH, NCH)], idx_v, isem)

    zrow = jnp.zeros((16,), jnp.float32)

    def _zero_row(i, carry):
        for cc in range(VPR):
            zbuf[i, pl.ds(cc * 16, 16)] = zrow
        return carry

    lax.fori_loop(0, RPT, _zero_row, 0)
    for k in range(K):
        pltpu.async_copy(zbuf, acc_sh.at[pl.ds(k * S + s * RPT, RPT)], zs)
    for k in range(K):
        pltpu.make_async_copy(zbuf, acc_sh.at[pl.ds(s * RPT, RPT)], zs).wait()
    pltpu.make_async_copy(b_hbm.at[pl.ds(0, NCH)], idx_v, isem).wait()
    plsc.subcore_barrier()

    def _grp(g, carry):
        for b in range(NBUF):
            j = g * NBUF + b
            bf = (b + NBUF - 1) % NBUF

            @pl.when(jnp.logical_and(j >= 1, j + NBUF - 1 < NCH))
            def _():
                _scat_wait(bf)
                _fetch(j + NBUF - 1, bf)

            _fetch_wait(b)
            _scat(j, b)
        return carry

    lax.fori_loop(0, NCH // NBUF, _grp, 0)
    _fetch_wait(0)
    _scat(NCH - 1, 0)
    for b in [1, 2, 3, 0]:
        _scat_wait(b)
    plsc.subcore_barrier()

    pltpu.sync_copy(acc_sh.at[pl.ds(s * RPT, RPT)], zbuf)
    _rep = [xb0, xb1]
    for k in range(1, K):
        rb = _rep[k % 2]
        pltpu.async_copy(acc_sh.at[pl.ds(k * S + s * RPT, RPT)],
                         rb.at[pl.ds(0, RPT)], fs[k % 2])
        if k > 1:
            pb = _rep[(k - 1) % 2]
            pltpu.make_async_copy(acc_sh.at[pl.ds(0, RPT)],
                                  pb.at[pl.ds(0, RPT)], fs[(k - 1) % 2]).wait()

            def _add_row(i, carry, _pb=pb):
                for cc in range(VPR):
                    zbuf[i, pl.ds(cc * 16, 16)] = (
                        zbuf[i, pl.ds(cc * 16, 16)]
                        + _pb[i, pl.ds(cc * 16, 16)])
                return carry

            lax.fori_loop(0, RPT, _add_row, 0)
    pltpu.make_async_copy(acc_sh.at[pl.ds(0, RPT)],
                          _rep[(K - 1) % 2].at[pl.ds(0, RPT)],
                          fs[(K - 1) % 2]).wait()

    def _add_last(i, carry):
        for cc in range(VPR):
            zbuf[i, pl.ds(cc * 16, 16)] = (
                zbuf[i, pl.ds(cc * 16, 16)]
                + _rep[(K - 1) % 2][i, pl.ds(cc * 16, 16)])
        return carry

    lax.fori_loop(0, RPT, _add_last, 0)
    pltpu.sync_copy(zbuf, out_hbm.at[c, pl.ds(s * RPT, RPT)])


_sc_call = functools.partial(
    pl.kernel,
    out_type=jax.ShapeDtypeStruct((NC, S, D), jnp.float32),
    mesh=plsc.VectorSubcoreMesh(core_axis_name="c", subcore_axis_name="s"),
    scratch_types=[
        pltpu.VMEM((NCH, C), jnp.int32),
        pltpu.VMEM((C, D), jnp.float32),
        pltpu.VMEM((C, D), jnp.float32),
        pltpu.VMEM((C, D), jnp.float32),
        pltpu.VMEM((C, D), jnp.float32),
        pltpu.VMEM((RPT, D), jnp.float32),
        pltpu.VMEM_SHARED((K * S, D), jnp.float32),
        pltpu.SemaphoreType.DMA,
        pltpu.SemaphoreType.DMA,
        pltpu.SemaphoreType.DMA,
        pltpu.SemaphoreType.DMA,
        pltpu.SemaphoreType.DMA,
        pltpu.SemaphoreType.DMA,
        pltpu.SemaphoreType.DMA,
        pltpu.SemaphoreType.DMA,
        pltpu.SemaphoreType.DMA,
        pltpu.SemaphoreType.DMA,
    ],
    compiler_params=pltpu.CompilerParams(use_tc_tiling_on_sc=False),
)(_sc_body)


def _combine_body(p_ref, o_ref):
    o_ref[...] = p_ref[0] + p_ref[1]


def kernel(x, batch):
    spread = (jnp.arange(C, dtype=jnp.int32) % K) * S
    b2 = batch.astype(jnp.int32).reshape(N // C, C) + spread[None, :]
    partials = _sc_call(x, b2)
    return pl.pallas_call(
        _combine_body,
        out_shape=jax.ShapeDtypeStruct((S, D), jnp.float32),
    )(partials)


# speedup vs baseline: 1.0782x; 1.0206x over previous
import functools

import jax
import jax.numpy as jnp
from jax import lax
from jax.experimental import pallas as pl
from jax.experimental.pallas import tpu as pltpu
from jax.experimental.pallas import tpu_sc as plsc

N = 100000
D = 128
S = 512
NC = 2
NS = 16
NW = NC * NS
RPW = N // NW
C = 125
NCH = RPW // C
K = 2
RPT = S // NS
VPR = D // 16


NBUF = 4


def _sc_body(x_hbm, b_hbm, out_hbm, idx_v, xb0, xb1, xb2, xb3, zbuf,
             acc_sh, fs0, fs1, fs2, fs3, ss0, ss1, ss2, ss3, zs, isem):
    c = lax.axis_index("c")
    s = lax.axis_index("s")
    wid = s * NC + c
    base = wid * RPW
    xb = [xb0, xb1, xb2, xb3]
    fs = [fs0, fs1, fs2, fs3]
    ss = [ss0, ss1, ss2, ss3]

    def _fetch(j, b):
        pltpu.async_copy(x_hbm.at[pl.ds(base + j * C, C)], xb[b], fs[b])

    def _fetch_wait(b):
        pltpu.make_async_copy(x_hbm.at[pl.ds(base, C)], xb[b], fs[b]).wait()

    def _scat(j, b):
        pltpu.async_copy(xb[b], acc_sh.at[idx_v.at[j]], ss[b], add=True)

    def _scat_wait(b):
        pltpu.make_async_copy(xb[b], acc_sh.at[idx_v.at[0]], ss[b]).wait()

    for b in range(NBUF):
        _fetch(b, b)
    pltpu.async_copy(b_hbm.at[pl.ds(wid * NCH, NCH)], idx_v, isem)

    zrow = jnp.zeros((16,), jnp.float32)

    def _zero_row(i, carry):
        for cc in range(VPR):
            zbuf[i, pl.ds(cc * 16, 16)] = zrow
        return carry

    lax.fori_loop(0, RPT, _zero_row, 0)
    for k in range(K):
        pltpu.async_copy(zbuf, acc_sh.at[pl.ds(k * S + s * RPT, RPT)], zs)
    for k in range(K):
        pltpu.make_async_copy(zbuf, acc_sh.at[pl.ds(s * RPT, RPT)], zs).wait()
    pltpu.make_async_copy(b_hbm.at[pl.ds(0, NCH)], idx_v, isem).wait()
    plsc.subcore_barrier()

    def _grp(g, carry):
        for b in range(NBUF):
            j = g * NBUF + b
            bf = (b + NBUF - 1) % NBUF

            @pl.when(jnp.logical_and(j >= 1, j + NBUF - 1 < NCH))
            def _():
                _scat_wait(bf)
                _fetch(j + NBUF - 1, bf)

            _fetch_wait(b)
            _scat(j, b)
        return carry

    lax.fori_loop(0, NCH // NBUF, _grp, 0)
    _fetch_wait(0)
    _scat(NCH - 1, 0)
    for b in [1, 2, 3, 0]:
        _scat_wait(b)
    plsc.subcore_barrier()

    pltpu.sync_copy(acc_sh.at[pl.ds(s * RPT, RPT)], zbuf)
    _rep = [xb0, xb1]
    for k in range(1, K):
        rb = _rep[k % 2]
        pltpu.async_copy(acc_sh.at[pl.ds(k * S + s * RPT, RPT)],
                         rb.at[pl.ds(0, RPT)], fs[k % 2])
        if k > 1:
            pb = _rep[(k - 1) % 2]
            pltpu.make_async_copy(acc_sh.at[pl.ds(0, RPT)],
                                  pb.at[pl.ds(0, RPT)], fs[(k - 1) % 2]).wait()

            def _add_row(i, carry, _pb=pb):
                for cc in range(VPR):
                    zbuf[i, pl.ds(cc * 16, 16)] = (
                        zbuf[i, pl.ds(cc * 16, 16)]
                        + _pb[i, pl.ds(cc * 16, 16)])
                return carry

            lax.fori_loop(0, RPT, _add_row, 0)
    pltpu.make_async_copy(acc_sh.at[pl.ds(0, RPT)],
                          _rep[(K - 1) % 2].at[pl.ds(0, RPT)],
                          fs[(K - 1) % 2]).wait()

    def _add_last(i, carry):
        for cc in range(VPR):
            zbuf[i, pl.ds(cc * 16, 16)] = (
                zbuf[i, pl.ds(cc * 16, 16)]
                + _rep[(K - 1) % 2][i, pl.ds(cc * 16, 16)])
        return carry

    lax.fori_loop(0, RPT, _add_last, 0)
    pltpu.sync_copy(zbuf, out_hbm.at[c, pl.ds(s * RPT, RPT)])


_sc_call = functools.partial(
    pl.kernel,
    out_type=jax.ShapeDtypeStruct((NC, S, D), jnp.float32),
    mesh=plsc.VectorSubcoreMesh(core_axis_name="c", subcore_axis_name="s"),
    scratch_types=[
        pltpu.VMEM((NCH, C), jnp.int32),
        pltpu.VMEM((C, D), jnp.float32),
        pltpu.VMEM((C, D), jnp.float32),
        pltpu.VMEM((C, D), jnp.float32),
        pltpu.VMEM((C, D), jnp.float32),
        pltpu.VMEM((RPT, D), jnp.float32),
        pltpu.VMEM_SHARED((K * S, D), jnp.float32),
        pltpu.SemaphoreType.DMA,
        pltpu.SemaphoreType.DMA,
        pltpu.SemaphoreType.DMA,
        pltpu.SemaphoreType.DMA,
        pltpu.SemaphoreType.DMA,
        pltpu.SemaphoreType.DMA,
        pltpu.SemaphoreType.DMA,
        pltpu.SemaphoreType.DMA,
        pltpu.SemaphoreType.DMA,
        pltpu.SemaphoreType.DMA,
    ],
    compiler_params=pltpu.CompilerParams(use_tc_tiling_on_sc=False),
)(_sc_body)


def _combine_body(p_ref, o_ref):
    o_ref[...] = p_ref[0] + p_ref[1]


def kernel(x, batch):
    spread = (jnp.arange(C, dtype=jnp.int32) % K) * S
    b2 = batch.astype(jnp.int32).reshape(N // C, C) + spread[None, :]
    partials = _sc_call(x, b2)
    return pl.pallas_call(
        _combine_body,
        out_shape=jax.ShapeDtypeStruct((S, D), jnp.float32),
    )(partials)


# speedup vs baseline: 1.0923x; 1.0130x over previous
import functools

import jax
import jax.numpy as jnp
from jax import lax
from jax.experimental import pallas as pl
from jax.experimental.pallas import tpu as pltpu
from jax.experimental.pallas import tpu_sc as plsc

N = 100000
D = 128
S = 512
NC = 2
NS = 16
NW = NC * NS
RPW = N // NW
C = 125
NCH = RPW // C
K = 1
RPT = S // NS
VPR = D // 16


NBUF = 4


def _sc_body(x_hbm, b_hbm, out_hbm, idx_v, xb0, xb1, xb2, xb3, zbuf,
             acc_sh, fs0, fs1, fs2, fs3, ss0, ss1, ss2, ss3, zs, isem):
    c = lax.axis_index("c")
    s = lax.axis_index("s")
    wid = s * NC + c
    base = wid * RPW
    xb = [xb0, xb1, xb2, xb3]
    fs = [fs0, fs1, fs2, fs3]
    ss = [ss0, ss1, ss2, ss3]

    def _fetch(j, b):
        pltpu.async_copy(x_hbm.at[pl.ds(base + j * C, C)], xb[b], fs[b])

    def _fetch_wait(b):
        pltpu.make_async_copy(x_hbm.at[pl.ds(base, C)], xb[b], fs[b]).wait()

    def _scat(j, b):
        pltpu.async_copy(xb[b], acc_sh.at[idx_v.at[j]], ss[b], add=True)

    def _scat_wait(b):
        pltpu.make_async_copy(xb[b], acc_sh.at[idx_v.at[0]], ss[b]).wait()

    for b in range(NBUF):
        _fetch(b, b)
    pltpu.async_copy(b_hbm.at[pl.ds(wid * NCH, NCH)], idx_v, isem)

    zrow = jnp.zeros((16,), jnp.float32)

    def _zero_row(i, carry):
        for cc in range(VPR):
            zbuf[i, pl.ds(cc * 16, 16)] = zrow
        return carry

    lax.fori_loop(0, RPT, _zero_row, 0)
    for k in range(K):
        pltpu.async_copy(zbuf, acc_sh.at[pl.ds(k * S + s * RPT, RPT)], zs)
    for k in range(K):
        pltpu.make_async_copy(zbuf, acc_sh.at[pl.ds(s * RPT, RPT)], zs).wait()
    pltpu.make_async_copy(b_hbm.at[pl.ds(0, NCH)], idx_v, isem).wait()
    plsc.subcore_barrier()

    def _grp(g, carry):
        for b in range(NBUF):
            j = g * NBUF + b
            bf = (b + NBUF - 1) % NBUF

            @pl.when(jnp.logical_and(j >= 1, j + NBUF - 1 < NCH))
            def _():
                _scat_wait(bf)
                _fetch(j + NBUF - 1, bf)

            _fetch_wait(b)
            _scat(j, b)
        return carry

    lax.fori_loop(0, NCH // NBUF, _grp, 0)
    _fetch_wait(0)
    _scat(NCH - 1, 0)
    for b in [1, 2, 3, 0]:
        _scat_wait(b)
    plsc.subcore_barrier()

    if K == 1:
        pltpu.sync_copy(acc_sh.at[pl.ds(s * RPT, RPT)],
                        out_hbm.at[c, pl.ds(s * RPT, RPT)])
        return
    pltpu.sync_copy(acc_sh.at[pl.ds(s * RPT, RPT)], zbuf)
    _rep = [xb0, xb1]
    for k in range(1, K):
        rb = _rep[k % 2]
        pltpu.async_copy(acc_sh.at[pl.ds(k * S + s * RPT, RPT)],
                         rb.at[pl.ds(0, RPT)], fs[k % 2])
        if k > 1:
            pb = _rep[(k - 1) % 2]
            pltpu.make_async_copy(acc_sh.at[pl.ds(0, RPT)],
                                  pb.at[pl.ds(0, RPT)], fs[(k - 1) % 2]).wait()

            def _add_row(i, carry, _pb=pb):
                for cc in range(VPR):
                    zbuf[i, pl.ds(cc * 16, 16)] = (
                        zbuf[i, pl.ds(cc * 16, 16)]
                        + _pb[i, pl.ds(cc * 16, 16)])
                return carry

            lax.fori_loop(0, RPT, _add_row, 0)
    pltpu.make_async_copy(acc_sh.at[pl.ds(0, RPT)],
                          _rep[(K - 1) % 2].at[pl.ds(0, RPT)],
                          fs[(K - 1) % 2]).wait()

    def _add_last(i, carry):
        for cc in range(VPR):
            zbuf[i, pl.ds(cc * 16, 16)] = (
                zbuf[i, pl.ds(cc * 16, 16)]
                + _rep[(K - 1) % 2][i, pl.ds(cc * 16, 16)])
        return carry

    lax.fori_loop(0, RPT, _add_last, 0)
    pltpu.sync_copy(zbuf, out_hbm.at[c, pl.ds(s * RPT, RPT)])


_sc_call = functools.partial(
    pl.kernel,
    out_type=jax.ShapeDtypeStruct((NC, S, D), jnp.float32),
    mesh=plsc.VectorSubcoreMesh(core_axis_name="c", subcore_axis_name="s"),
    scratch_types=[
        pltpu.VMEM((NCH, C), jnp.int32),
        pltpu.VMEM((C, D), jnp.float32),
        pltpu.VMEM((C, D), jnp.float32),
        pltpu.VMEM((C, D), jnp.float32),
        pltpu.VMEM((C, D), jnp.float32),
        pltpu.VMEM((RPT, D), jnp.float32),
        pltpu.VMEM_SHARED((K * S, D), jnp.float32),
        pltpu.SemaphoreType.DMA,
        pltpu.SemaphoreType.DMA,
        pltpu.SemaphoreType.DMA,
        pltpu.SemaphoreType.DMA,
        pltpu.SemaphoreType.DMA,
        pltpu.SemaphoreType.DMA,
        pltpu.SemaphoreType.DMA,
        pltpu.SemaphoreType.DMA,
        pltpu.SemaphoreType.DMA,
        pltpu.SemaphoreType.DMA,
    ],
    compiler_params=pltpu.CompilerParams(use_tc_tiling_on_sc=False),
)(_sc_body)


def _combine_body(p_ref, o_ref):
    o_ref[...] = p_ref[0] + p_ref[1]


def kernel(x, batch):
    spread = (jnp.arange(C, dtype=jnp.int32) % K) * S
    b2 = batch.astype(jnp.int32).reshape(N // C, C) + spread[None, :]
    partials = _sc_call(x, b2)
    return pl.pallas_call(
        _combine_body,
        out_shape=jax.ShapeDtypeStruct((S, D), jnp.float32),
    )(partials)


# speedup vs baseline: 1.0924x; 1.0001x over previous
import functools

import jax
import jax.numpy as jnp
from jax import lax
from jax.experimental import pallas as pl
from jax.experimental.pallas import tpu as pltpu
from jax.experimental.pallas import tpu_sc as plsc

N = 100000
D = 128
S = 512
NC = 2
NS = 16
NW = NC * NS
RPW = N // NW
C = 125
NCH = RPW // C
RPT = S // NS
VPR = D // 16
NBUF = 4


def _sc_body(x_hbm, b_hbm, out_hbm, idx_v, xb0, xb1, xb2, xb3, zbuf,
             acc_sh, fs0, fs1, fs2, fs3, ss0, ss1, ss2, ss3, zs, isem):
    c = lax.axis_index("c")
    s = lax.axis_index("s")
    wid = s * NC + c
    base = wid * RPW
    xb = [xb0, xb1, xb2, xb3]
    fs = [fs0, fs1, fs2, fs3]
    ss = [ss0, ss1, ss2, ss3]

    def _fetch(j, b):
        pltpu.async_copy(x_hbm.at[pl.ds(base + j * C, C)], xb[b], fs[b])

    def _fetch_wait(b):
        pltpu.make_async_copy(x_hbm.at[pl.ds(base, C)], xb[b], fs[b]).wait()

    def _scat(j, b):
        pltpu.async_copy(xb[b], acc_sh.at[idx_v.at[j]], ss[b], add=True)

    def _scat_wait(b):
        pltpu.make_async_copy(xb[b], acc_sh.at[idx_v.at[0]], ss[b]).wait()

    for b in range(NBUF):
        _fetch(b, b)
    pltpu.async_copy(b_hbm.at[pl.ds(wid * NCH, NCH)], idx_v, isem)

    zrow = jnp.zeros((16,), jnp.float32)

    def _zero_row(i, carry):
        for cc in range(VPR):
            zbuf[i, pl.ds(cc * 16, 16)] = zrow
        return carry

    lax.fori_loop(0, RPT, _zero_row, 0)
    pltpu.async_copy(zbuf, acc_sh.at[pl.ds(s * RPT, RPT)], zs)
    pltpu.make_async_copy(zbuf, acc_sh.at[pl.ds(s * RPT, RPT)], zs).wait()
    pltpu.make_async_copy(b_hbm.at[pl.ds(0, NCH)], idx_v, isem).wait()
    plsc.subcore_barrier()

    def _grp(g, carry):
        for b in range(NBUF):
            j = g * NBUF + b
            bf = (b + NBUF - 1) % NBUF

            @pl.when(jnp.logical_and(j >= 1, j + NBUF - 1 < NCH))
            def _():
                _scat_wait(bf)
                _fetch(j + NBUF - 1, bf)

            _fetch_wait(b)
            _scat(j, b)
        return carry

    lax.fori_loop(0, NCH // NBUF, _grp, 0)
    _fetch_wait(0)
    _scat(NCH - 1, 0)
    for b in [1, 2, 3, 0]:
        _scat_wait(b)
    plsc.subcore_barrier()

    pltpu.sync_copy(acc_sh.at[pl.ds(s * RPT, RPT)],
                    out_hbm.at[c, pl.ds(s * RPT, RPT)])


_sc_call = functools.partial(
    pl.kernel,
    out_type=jax.ShapeDtypeStruct((NC, S, D), jnp.float32),
    mesh=plsc.VectorSubcoreMesh(core_axis_name="c", subcore_axis_name="s"),
    scratch_types=[
        pltpu.VMEM((NCH, C), jnp.int32),
        pltpu.VMEM((C, D), jnp.float32),
        pltpu.VMEM((C, D), jnp.float32),
        pltpu.VMEM((C, D), jnp.float32),
        pltpu.VMEM((C, D), jnp.float32),
        pltpu.VMEM((RPT, D), jnp.float32),
        pltpu.VMEM_SHARED((S, D), jnp.float32),
        pltpu.SemaphoreType.DMA,
        pltpu.SemaphoreType.DMA,
        pltpu.SemaphoreType.DMA,
        pltpu.SemaphoreType.DMA,
        pltpu.SemaphoreType.DMA,
        pltpu.SemaphoreType.DMA,
        pltpu.SemaphoreType.DMA,
        pltpu.SemaphoreType.DMA,
        pltpu.SemaphoreType.DMA,
        pltpu.SemaphoreType.DMA,
    ],
    compiler_params=pltpu.CompilerParams(use_tc_tiling_on_sc=False),
)(_sc_body)


def _combine_body(p_ref, o_ref):
    o_ref[...] = p_ref[0] + p_ref[1]


def kernel(x, batch):
    b2 = batch.astype(jnp.int32).reshape(N // C, C)
    partials = _sc_call(x, b2)
    return pl.pallas_call(
        _combine_body,
        out_shape=jax.ShapeDtypeStruct((S, D), jnp.float32),
    )(partials)
